# Initial kernel scaffold; baseline (speedup 1.0000x reference)
import jax
import jax.numpy as jnp
from jax.experimental import pallas as pl


def kernel(z, W_in, b_in, W_out, b_out, codebook):
    raise NotImplementedError("write your pallas kernel here")


# speedup vs baseline: 1.2292x; 1.2292x over previous
import functools

import jax
import jax.numpy as jnp
from jax import lax
from jax.experimental import pallas as pl


_PZE = lax.Precision.DEFAULT
_PDIST = lax.Precision.DEFAULT


def _encode_kernel(z_ref, w_ref, b_ref, cbnT_ref, cbsq_ref, ze_ref, idx_ref, *, Tt, K, Ck):
    zt = z_ref[0]
    w = w_ref[...]
    ze8t = jnp.dot(w, zt, preferred_element_type=jnp.float32,
                   precision=_PZE)
    ze = ze8t.T + b_ref[...]
    ze_ref[0] = ze

    nC = ze.shape[1]
    zsq = ze * ze
    nsq = zsq[:, 0:1]
    for j in range(1, nC):
        nsq = nsq + zsq[:, j:j + 1]
    n = jnp.sqrt(nsq)
    enc = ze / jnp.maximum(n, 1e-12)
    esq = enc * enc
    encsq = esq[:, 0:1]
    for j in range(1, nC):
        encsq = encsq + esq[:, j:j + 1]

    cbsqn = cbsq_ref[...]
    encb = enc
    cbn2 = cbnT_ref[...]

    bestv = jnp.full((Tt, 1), 1e30, jnp.float32)
    besti = jnp.zeros((Tt, 1), jnp.int32)
    for c in range(K // Ck):
        sl = slice(c * Ck, (c + 1) * Ck)
        dot2 = jnp.dot(encb, cbn2[:, sl], preferred_element_type=jnp.float32,
                       precision=_PDIST)
        dist = (encsq + dot2) + cbsqn[:, sl]
        cmin = jnp.min(dist, axis=1, keepdims=True)
        eq = dist == cmin
        io = lax.broadcasted_iota(jnp.int32, (Tt, Ck), 1)
        ci = jnp.min(jnp.where(eq, io, Ck), axis=1, keepdims=True) + c * Ck
        upd = cmin < bestv
        besti = jnp.where(upd, ci, besti)
        bestv = jnp.where(upd, cmin, bestv)
    idx_ref[0] = besti


def _decode_kernel(ze_ref, zq_ref, w_ref, b_ref, qo_ref, aq_ref, acc_ref):
    ze = ze_ref[0]
    zq = zq_ref[0]
    w = w_ref[...]
    out = jnp.dot(w, zq.T, preferred_element_type=jnp.float32,
                  precision=lax.Precision.HIGHEST) + b_ref[...]
    qo_ref[0] = out
    aq_ref[0, 0] = out

    @pl.when(jnp.logical_and(pl.program_id(0) == 0, pl.program_id(1) == 0))
    def _init():
        acc_ref[...] = jnp.zeros((1, 1), jnp.float32)

    d = ze - zq
    acc_ref[...] += jnp.sum(d * d).reshape(1, 1)


def kernel(z, W_in, b_in, W_out, b_out, codebook):
    B, D, T = z.shape
    K, C = codebook.shape
    Tt = 256
    nT = T // Tt

    ncb = jnp.sqrt(jnp.sum(codebook * codebook, axis=1, keepdims=True))
    cb_n = codebook / jnp.maximum(ncb, 1e-12)
    cbsqn = jnp.sum(cb_n * cb_n, axis=1, keepdims=True)
    cbn2 = -2.0 * cb_n

    z_eT, idx3 = pl.pallas_call(
        functools.partial(_encode_kernel, Tt=Tt, K=K, Ck=2048),
        grid=(B, nT),
        in_specs=[
            pl.BlockSpec((1, D, Tt), lambda b, t: (b, 0, t)),
            pl.BlockSpec((C, D), lambda b, t: (0, 0)),
            pl.BlockSpec((1, C), lambda b, t: (0, 0)),
            pl.BlockSpec((C, K), lambda b, t: (0, 0)),
            pl.BlockSpec((1, K), lambda b, t: (0, 0)),
        ],
        out_specs=[
            pl.BlockSpec((1, Tt, C), lambda b, t: (b, t, 0)),
            pl.BlockSpec((1, Tt, 1), lambda b, t: (b, t, 0)),
        ],
        out_shape=[
            jax.ShapeDtypeStruct((B, T, C), jnp.float32),
            jax.ShapeDtypeStruct((B, T, 1), jnp.int32),
        ],
    )(z, W_in, b_in.reshape(1, C), cbn2.T, cbsqn.reshape(1, K))

    idx_flat = idx3.reshape(B * T)
    zq = jnp.take(codebook, idx_flat, axis=0).reshape(B, T, C)

    qo, aq, acc = pl.pallas_call(
        _decode_kernel,
        grid=(B, nT),
        in_specs=[
            pl.BlockSpec((1, Tt, C), lambda b, t: (b, t, 0)),
            pl.BlockSpec((1, Tt, C), lambda b, t: (b, t, 0)),
            pl.BlockSpec((D, C), lambda b, t: (0, 0)),
            pl.BlockSpec((D, 1), lambda b, t: (0, 0)),
        ],
        out_specs=[
            pl.BlockSpec((1, D, Tt), lambda b, t: (b, 0, t)),
            pl.BlockSpec((1, 1, D, Tt), lambda b, t: (0, b, 0, t)),
            pl.BlockSpec((1, 1), lambda b, t: (0, 0)),
        ],
        out_shape=[
            jax.ShapeDtypeStruct((B, D, T), jnp.float32),
            jax.ShapeDtypeStruct((1, B, D, T), jnp.float32),
            jax.ShapeDtypeStruct((1, 1), jnp.float32),
        ],
    )(z_eT, zq, W_out, b_out.reshape(D, 1))

    loss = (acc[0, 0] / jnp.float32(B * C * T))[None]
    all_indices = idx3.reshape(B, T)[None]
    return (qo, all_indices, loss, loss, aq)
